# baseline (speedup 1.0000x reference)
import functools

import jax
import jax.numpy as jnp
from jax import lax
from jax.experimental import pallas as pl
from jax.experimental.pallas import tpu as pltpu
from jax.experimental.pallas import tpu_sc as plsc

N_NODES = 10000
D = 128
N_EDGES = 320000
BN_EPS = 1e-5

NC = 2
NS = 16
NW = NC * NS
CHK = 128
CPW = 80
EPAD = NW * CPW * CHK
NPB = 632
NPAD = NS * NPB

_mesh = plsc.VectorSubcoreMesh(core_axis_name="c", subcore_axis_name="s")

NBUF = 2
HB = 40


@functools.partial(
    pl.kernel,
    out_type=jax.ShapeDtypeStruct((NC, NPAD, D), jnp.float32),
    mesh=_mesh,
    scratch_types=[
        pltpu.VMEM((HB, CHK), jnp.int32),
        pltpu.VMEM((HB, CHK), jnp.int32),
        pltpu.VMEM((NBUF, CHK, D), jnp.float32),
        pltpu.VMEM_SHARED((NPAD, D), jnp.float32),
        pltpu.SemaphoreType.DMA((NBUF,)),
        pltpu.SemaphoreType.DMA((NBUF,)),
    ],
)
def _sc_aggregate(h_hbm, ei_hbm, z_hbm, out_hbm, src_v, dst_v, rows_v,
                  acc_sh, gsem, ssem):
    c = lax.axis_index("c")
    s = lax.axis_index("s")
    w = s * NC + c
    with jax.named_scope("init_h"):
        @pl.when(c == 0)
        def _():
            pltpu.sync_copy(h_hbm.at[pl.ds(s * NPB, NPB)],
                            acc_sh.at[pl.ds(s * NPB, NPB)])

        @pl.when(c != 0)
        def _():
            pltpu.sync_copy(z_hbm.at[pl.ds(s * NPB, NPB)],
                            acc_sh.at[pl.ds(s * NPB, NPB)])

        plsc.subcore_barrier()

    for half in range(CPW // HB):
      with jax.named_scope(f"edges{half}"):
          base = w * CPW + half * HB
          pltpu.sync_copy(ei_hbm.at[0, pl.ds(base, HB)], src_v)
          pltpu.sync_copy(ei_hbm.at[1, pl.ds(base, HB)], dst_v)
          for b in range(NBUF):
              pltpu.async_copy(h_hbm.at[src_v.at[b]], rows_v.at[b], gsem.at[b])

          @pl.loop(0, HB - NBUF, step=NBUF)
          def _steady(g):
              descs = []
              for b in range(NBUF):
                  j = g + b
                  pltpu.make_async_copy(h_hbm.at[src_v.at[j]], rows_v.at[b],
                                        gsem.at[b]).wait()
                  descs.append(pltpu.async_copy(
                      rows_v.at[b], acc_sh.at[dst_v.at[j]], ssem.at[b], add=True))
              for b in range(NBUF):
                  descs[b].wait()
                  pltpu.async_copy(h_hbm.at[src_v.at[g + NBUF + b]], rows_v.at[b],
                                   gsem.at[b])

          descs = []
          for b in range(NBUF):
              j = HB - NBUF + b
              pltpu.make_async_copy(h_hbm.at[src_v.at[j]], rows_v.at[b],
                                    gsem.at[b]).wait()
              descs.append(pltpu.async_copy(
                  rows_v.at[b], acc_sh.at[dst_v.at[j]], ssem.at[b], add=True))
          for d_ in descs:
              d_.wait()

    with jax.named_scope("copyout"):
        plsc.subcore_barrier()
        pltpu.sync_copy(acc_sh.at[pl.ds(s * NPB, NPB)],
                        out_hbm.at[c, pl.ds(s * NPB, NPB)])


def _mlp_body(last, p0_ref, p1_ref, w1_ref, b1_ref, g_ref, bt_ref,
              w2_ref, b2_ref, o_ref):
    z = p0_ref[0] + p1_ref[0]
    z = jnp.dot(z, w1_ref[...], preferred_element_type=jnp.float32) + b1_ref[...]
    z = jnp.maximum(z, 0.0)
    scale = g_ref[...] * (1.0 / jnp.sqrt(1.0 + BN_EPS))
    z = z * scale + bt_ref[...]
    z = jnp.dot(z, w2_ref[...], preferred_element_type=jnp.float32) + b2_ref[...]
    if last:
        m = jnp.max(z, axis=-1, keepdims=True)
        z = z - (m + jnp.log(jnp.sum(jnp.exp(z - m), axis=-1, keepdims=True)))
    else:
        z = jnp.maximum(z, 0.0)
    o_ref[...] = z


def _mlp(p, w1, b1, g, bt, w2, b2, last):
    blk = 1024
    grid = (NPAD + blk - 1) // blk
    row = pl.BlockSpec((blk, D), lambda i: (i, 0))
    p0s = pl.BlockSpec((1, blk, D), lambda i: (0, i, 0))
    p1s = pl.BlockSpec((1, blk, D), lambda i: (1, i, 0))
    full = pl.BlockSpec((1, D), lambda i: (0, 0))
    return pl.pallas_call(
        functools.partial(_mlp_body, last),
        grid=(grid,),
        in_specs=[p0s, p1s, pl.BlockSpec((D, D), lambda i: (0, 0)), full,
                  full, full, pl.BlockSpec((D, D), lambda i: (0, 0)), full],
        out_specs=row,
        out_shape=jax.ShapeDtypeStruct((NPAD, D), jnp.float32),
    )(p, p, w1, b1.reshape(1, D), g.reshape(1, D), bt.reshape(1, D),
      w2, b2.reshape(1, D))


def kernel(x, edge_index, params):
    ppw = (EPAD - N_EDGES) // NW
    pad_src = (jnp.arange(NW * ppw, dtype=jnp.int32) * 97) % N_NODES
    pad_dst = N_NODES + jnp.arange(NW * ppw, dtype=jnp.int32) % (NPAD - N_NODES)
    pad_block = jnp.stack([pad_src, pad_dst]).reshape(2, NW, ppw)
    ei = jnp.concatenate(
        [edge_index.astype(jnp.int32).reshape(2, NW, -1), pad_block],
        axis=2).reshape(2, -1, CHK)
    h = jnp.pad(x, ((0, NPAD - N_NODES), (0, 0)))
    zeros = jnp.zeros((NPAD, D), jnp.float32)
    n_layers = len(params)
    for i, (w1, b1, g, bt, w2, b2) in enumerate(params):
        p = _sc_aggregate(h, ei, zeros)
        h = _mlp(p, w1, b1, g, bt, w2, b2, i == n_layers - 1)
    return h[:N_NODES]


# speedup vs baseline: 1.0305x; 1.0305x over previous
import functools

import jax
import jax.numpy as jnp
from jax import lax
from jax.experimental import pallas as pl
from jax.experimental.pallas import tpu as pltpu
from jax.experimental.pallas import tpu_sc as plsc

N_NODES = 10000
D = 128
N_EDGES = 320000
BN_EPS = 1e-5

NC = 2
NS = 16
NW = NC * NS
CHK = 128
CPW = 80
EPAD = NW * CPW * CHK
NPB = 632
NPAD = NS * NPB

_mesh = plsc.VectorSubcoreMesh(core_axis_name="c", subcore_axis_name="s")

NBUF = 2
HB = 40


@functools.partial(
    pl.kernel,
    out_type=jax.ShapeDtypeStruct((NC, NPAD, D), jnp.float32),
    mesh=_mesh,
    scratch_types=[
        pltpu.VMEM((HB, CHK), jnp.int32),
        pltpu.VMEM((HB, CHK), jnp.int32),
        pltpu.VMEM((NBUF, CHK, D), jnp.float32),
        pltpu.VMEM_SHARED((NPAD, D), jnp.float32),
        pltpu.SemaphoreType.DMA((NBUF,)),
        pltpu.SemaphoreType.DMA((NBUF,)),
        pltpu.SemaphoreType.DMA,
    ],
)
def _sc_aggregate(h_hbm, ei_hbm, z_hbm, out_hbm, src_v, dst_v, rows_v,
                  acc_sh, gsem, ssem, isem):
    c = lax.axis_index("c")
    s = lax.axis_index("s")
    w = s * NC + c
    with jax.named_scope("init_h"):
        @pl.when(c == 0)
        def _():
            pltpu.async_copy(h_hbm.at[pl.ds(s * NPB, NPB)],
                             acc_sh.at[pl.ds(s * NPB, NPB)], isem)

        @pl.when(c != 0)
        def _():
            pltpu.async_copy(z_hbm.at[pl.ds(s * NPB, NPB)],
                             acc_sh.at[pl.ds(s * NPB, NPB)], isem)

    for half in range(CPW // HB):
      with jax.named_scope(f"edges{half}"):
          base = w * CPW + half * HB
          pltpu.sync_copy(ei_hbm.at[0, pl.ds(base, HB)], src_v)
          pltpu.sync_copy(ei_hbm.at[1, pl.ds(base, HB)], dst_v)
          for b in range(NBUF):
              pltpu.async_copy(h_hbm.at[src_v.at[b]], rows_v.at[b], gsem.at[b])
          if half == 0:
              pltpu.make_async_copy(h_hbm.at[pl.ds(s * NPB, NPB)],
                                    acc_sh.at[pl.ds(s * NPB, NPB)], isem).wait()
              plsc.subcore_barrier()

          @pl.loop(0, HB - NBUF, step=NBUF)
          def _steady(g):
              descs = []
              for b in range(NBUF):
                  j = g + b
                  pltpu.make_async_copy(h_hbm.at[src_v.at[j]], rows_v.at[b],
                                        gsem.at[b]).wait()
                  descs.append(pltpu.async_copy(
                      rows_v.at[b], acc_sh.at[dst_v.at[j]], ssem.at[b], add=True))
              for b in range(NBUF):
                  descs[b].wait()
                  pltpu.async_copy(h_hbm.at[src_v.at[g + NBUF + b]], rows_v.at[b],
                                   gsem.at[b])

          descs = []
          for b in range(NBUF):
              j = HB - NBUF + b
              pltpu.make_async_copy(h_hbm.at[src_v.at[j]], rows_v.at[b],
                                    gsem.at[b]).wait()
              descs.append(pltpu.async_copy(
                  rows_v.at[b], acc_sh.at[dst_v.at[j]], ssem.at[b], add=True))
          for d_ in descs:
              d_.wait()

    with jax.named_scope("copyout"):
        plsc.subcore_barrier()
        pltpu.sync_copy(acc_sh.at[pl.ds(s * NPB, NPB)],
                        out_hbm.at[c, pl.ds(s * NPB, NPB)])


def _mlp_body(last, p0_ref, p1_ref, w1_ref, b1_ref, g_ref, bt_ref,
              w2_ref, b2_ref, o_ref):
    z = p0_ref[0] + p1_ref[0]
    z = jnp.dot(z, w1_ref[...], preferred_element_type=jnp.float32) + b1_ref[...]
    z = jnp.maximum(z, 0.0)
    scale = g_ref[...] * (1.0 / jnp.sqrt(1.0 + BN_EPS))
    z = z * scale + bt_ref[...]
    z = jnp.dot(z, w2_ref[...], preferred_element_type=jnp.float32) + b2_ref[...]
    if last:
        m = jnp.max(z, axis=-1, keepdims=True)
        z = z - (m + jnp.log(jnp.sum(jnp.exp(z - m), axis=-1, keepdims=True)))
    else:
        z = jnp.maximum(z, 0.0)
    o_ref[...] = z


def _mlp(p, w1, b1, g, bt, w2, b2, last):
    blk = 1024
    grid = (NPAD + blk - 1) // blk
    out_rows = N_NODES if last else NPAD
    row = pl.BlockSpec((blk, D), lambda i: (i, 0))
    p0s = pl.BlockSpec((1, blk, D), lambda i: (0, i, 0))
    p1s = pl.BlockSpec((1, blk, D), lambda i: (1, i, 0))
    full = pl.BlockSpec((1, D), lambda i: (0, 0))
    return pl.pallas_call(
        functools.partial(_mlp_body, last),
        grid=(grid,),
        in_specs=[p0s, p1s, pl.BlockSpec((D, D), lambda i: (0, 0)), full,
                  full, full, pl.BlockSpec((D, D), lambda i: (0, 0)), full],
        out_specs=row,
        out_shape=jax.ShapeDtypeStruct((out_rows, D), jnp.float32),
    )(p, p, w1, b1.reshape(1, D), g.reshape(1, D), bt.reshape(1, D),
      w2, b2.reshape(1, D))


def kernel(x, edge_index, params):
    ppw = (EPAD - N_EDGES) // NW
    pad_src = (jnp.arange(NW * ppw, dtype=jnp.int32) * 97) % N_NODES
    pad_dst = N_NODES + jnp.arange(NW * ppw, dtype=jnp.int32) % (NPAD - N_NODES)
    pad_block = jnp.stack([pad_src, pad_dst]).reshape(2, NW, ppw)
    ei = jnp.concatenate(
        [edge_index.astype(jnp.int32).reshape(2, NW, -1), pad_block],
        axis=2).reshape(2, -1, CHK)
    h = jnp.pad(x, ((0, NPAD - N_NODES), (0, 0)))
    zeros = jnp.zeros((NPAD, D), jnp.float32)
    n_layers = len(params)
    for i, (w1, b1, g, bt, w2, b2) in enumerate(params):
        p = _sc_aggregate(h, ei, zeros)
        h = _mlp(p, w1, b1, g, bt, w2, b2, i == n_layers - 1)
    return h
